# Initial kernel scaffold; baseline (speedup 1.0000x reference)
import jax
import jax.numpy as jnp
from jax.experimental import pallas as pl


def kernel(dg_node_feat_discrete, lg_node_feat_continuous, lg_node_feat_discrete, dg_edge_feat, lg_edge_feat, dg_edge_index, lg_edge_index, lg_edge_atom, dg_node_seg, lg_node_seg, params):
    raise NotImplementedError("write your pallas kernel here")


# speedup vs baseline: 1.6766x; 1.6766x over previous
import functools
import numpy as np
import jax
import jax.numpy as jnp
from jax import lax
from jax.experimental import pallas as pl

_CUT_R = 5.0


def _bn(n, cap=4096):
    best = None
    for b in range(8, cap + 1, 8):
        if n % b == 0:
            best = b
    return best if best is not None else n


def _linrelu_body(nx, *refs):
    xs = refs[:nx]
    w_ref, b_ref, o_ref = refs[nx], refs[nx + 1], refs[nx + 2]
    acc = xs[0][...]
    for r in xs[1:]:
        acc = acc + r[...]
    y = jnp.dot(acc, w_ref[...], preferred_element_type=jnp.float32)
    o_ref[...] = jax.nn.relu(y + b_ref[...])


def _linrelu(xs, W, b):
    n, k = xs[0].shape
    h = W.shape[1]
    bn = _bn(n)
    grid = (n // bn,)
    in_specs = [pl.BlockSpec((bn, k), lambda i: (i, 0)) for _ in xs]
    in_specs += [pl.BlockSpec((k, h), lambda i: (0, 0)),
                 pl.BlockSpec((1, h), lambda i: (0, 0))]
    return pl.pallas_call(
        functools.partial(_linrelu_body, len(xs)),
        grid=grid,
        in_specs=in_specs,
        out_specs=pl.BlockSpec((bn, h), lambda i: (i, 0)),
        out_shape=jax.ShapeDtypeStruct((n, h), jnp.float32),
    )(*xs, W, b.reshape(1, h))


def _rbf_body(kind, r_dim, d_ref, w_ref, b_ref, o_ref):
    d = d_ref[...]
    bn = d.shape[0]
    it = lax.broadcasted_iota(jnp.int32, (bn, r_dim), 1).astype(jnp.float32)
    if kind == 2:
        mu = it * (np.pi / (r_dim - 1))
        gamma = 8.0
    else:
        mu = it * (_CUT_R / (r_dim - 1))
        gamma = 10.0
    feat = jnp.exp(-gamma * (d - mu) ** 2)
    if kind == 1:
        env = 0.5 * (jnp.cos(np.pi * jnp.clip(d, 0.0, _CUT_R) / _CUT_R) + 1.0)
        feat = feat * env
    y = jnp.dot(feat, w_ref[...], preferred_element_type=jnp.float32)
    o_ref[...] = jax.nn.relu(y + b_ref[...])


def _edge_module(d, W, b, kind):
    n = d.shape[0]
    r_dim, h = W.shape
    bn = _bn(n)
    return pl.pallas_call(
        functools.partial(_rbf_body, kind, r_dim),
        grid=(n // bn,),
        in_specs=[pl.BlockSpec((bn, 1), lambda i: (i, 0)),
                  pl.BlockSpec((r_dim, h), lambda i: (0, 0)),
                  pl.BlockSpec((1, h), lambda i: (0, 0))],
        out_specs=pl.BlockSpec((bn, h), lambda i: (i, 0)),
        out_shape=jax.ShapeDtypeStruct((n, h), jnp.float32),
    )(d, W, b.reshape(1, h))


def _dgnode_body(t_dim, disc_ref, emb_ref, w_ref, b_ref, o_ref):
    disc = disc_ref[...]
    bn = disc.shape[0]
    it = lax.broadcasted_iota(jnp.int32, (bn, t_dim), 1)
    oh = (disc == it).astype(jnp.float32)
    e = jnp.dot(oh, emb_ref[...], preferred_element_type=jnp.float32)
    y = jnp.dot(e, w_ref[...], preferred_element_type=jnp.float32)
    o_ref[...] = jax.nn.relu(y + b_ref[...])


def _dg_node_module(disc, emb, W, b):
    n = disc.shape[0]
    t_dim, h = emb.shape
    bn = _bn(n)
    return pl.pallas_call(
        functools.partial(_dgnode_body, t_dim),
        grid=(n // bn,),
        in_specs=[pl.BlockSpec((bn, 1), lambda i: (i, 0)),
                  pl.BlockSpec((t_dim, h), lambda i: (0, 0)),
                  pl.BlockSpec((h, h), lambda i: (0, 0)),
                  pl.BlockSpec((1, h), lambda i: (0, 0))],
        out_specs=pl.BlockSpec((bn, h), lambda i: (i, 0)),
        out_shape=jax.ShapeDtypeStruct((n, h), jnp.float32),
    )(disc.reshape(n, 1), emb, W, b.reshape(1, h))


def _lgnode_body(t_dim, r_dim, disc_ref, cont_ref, emb_ref, w1_ref, w2_ref,
                 b_ref, o_ref):
    disc = disc_ref[...]
    bn = disc.shape[0]
    it = lax.broadcasted_iota(jnp.int32, (bn, t_dim), 1)
    oh = (disc == it).astype(jnp.float32)
    e = jnp.dot(oh, emb_ref[...], preferred_element_type=jnp.float32)
    d = cont_ref[...]
    itf = lax.broadcasted_iota(jnp.int32, (bn, r_dim), 1).astype(jnp.float32)
    mu = itf * (_CUT_R / (r_dim - 1))
    feat = jnp.exp(-10.0 * (d - mu) ** 2)
    y = (jnp.dot(e, w1_ref[...], preferred_element_type=jnp.float32)
         + jnp.dot(feat, w2_ref[...], preferred_element_type=jnp.float32))
    o_ref[...] = jax.nn.relu(y + b_ref[...])


def _lg_node_module(disc, cont, emb, W, b):
    n = disc.shape[0]
    t_dim, h = emb.shape
    r_dim = W.shape[0] - h
    bn = _bn(n)
    return pl.pallas_call(
        functools.partial(_lgnode_body, t_dim, r_dim),
        grid=(n // bn,),
        in_specs=[pl.BlockSpec((bn, 1), lambda i: (i, 0)),
                  pl.BlockSpec((bn, 1), lambda i: (i, 0)),
                  pl.BlockSpec((t_dim, h), lambda i: (0, 0)),
                  pl.BlockSpec((h, h), lambda i: (0, 0)),
                  pl.BlockSpec((r_dim, h), lambda i: (0, 0)),
                  pl.BlockSpec((1, h), lambda i: (0, 0))],
        out_specs=pl.BlockSpec((bn, h), lambda i: (i, 0)),
        out_shape=jax.ShapeDtypeStruct((n, h), jnp.float32),
    )(disc.reshape(n, 1), cont, emb, W[:h], W[h:], b.reshape(1, h))


def _readout_body(t_dim, n_graphs, h_ref, disc_ref, seg_ref, wo_ref, sct_ref,
                  bit_ref, cons_ref, score_ref, feat_ref):
    i = pl.program_id(0)
    h = h_ref[...]
    bn = h.shape[0]
    disc = disc_ref[...]
    it = lax.broadcasted_iota(jnp.int32, (bn, t_dim), 1)
    oh = (disc == it).astype(jnp.float32)
    node_out = jnp.sum(h * wo_ref[...], axis=1, keepdims=True)
    scv = jnp.sum(oh * sct_ref[...], axis=1, keepdims=True)
    biv = jnp.sum(oh * bit_ref[...], axis=1, keepdims=True)
    bo = cons_ref[0, 0]
    std = cons_ref[0, 1]
    mean = cons_ref[0, 2]
    node_out = scv * (node_out + bo) + biv
    node_out = node_out * std + mean
    seg = seg_ref[...]
    itg = lax.broadcasted_iota(jnp.int32, (bn, n_graphs), 1)
    ohs = (seg == itg).astype(jnp.float32)
    part_score = lax.dot_general(ohs, node_out, (((0,), (0,)), ((), ())),
                                 preferred_element_type=jnp.float32)
    part_feat = lax.dot_general(ohs, h, (((0,), (0,)), ((), ())),
                                preferred_element_type=jnp.float32)

    @pl.when(i == 0)
    def _():
        score_ref[...] = jnp.zeros_like(score_ref)
        feat_ref[...] = jnp.zeros_like(feat_ref)

    score_ref[...] += part_score
    feat_ref[...] += part_feat


def _readout(h, disc, seg, Wo, bo, sc, bi, mean, std, n_graphs):
    n, hd = h.shape
    t_dim = sc.shape[0]
    bn = _bn(n)
    cons = jnp.stack([bo[0], std[0], mean[0]]).reshape(1, 3)
    score, feat = pl.pallas_call(
        functools.partial(_readout_body, t_dim, n_graphs),
        grid=(n // bn,),
        in_specs=[pl.BlockSpec((bn, hd), lambda i: (i, 0)),
                  pl.BlockSpec((bn, 1), lambda i: (i, 0)),
                  pl.BlockSpec((bn, 1), lambda i: (i, 0)),
                  pl.BlockSpec((1, hd), lambda i: (0, 0)),
                  pl.BlockSpec((1, t_dim), lambda i: (0, 0)),
                  pl.BlockSpec((1, t_dim), lambda i: (0, 0)),
                  pl.BlockSpec((1, 3), lambda i: (0, 0))],
        out_specs=[pl.BlockSpec((n_graphs, 1), lambda i: (0, 0)),
                   pl.BlockSpec((n_graphs, hd), lambda i: (0, 0))],
        out_shape=[jax.ShapeDtypeStruct((n_graphs, 1), jnp.float32),
                   jax.ShapeDtypeStruct((n_graphs, hd), jnp.float32)],
    )(h, disc.reshape(n, 1), seg.reshape(n, 1), Wo.reshape(1, hd),
      sc.reshape(1, t_dim), bi.reshape(1, t_dim), cons)
    return feat, score


def _fusion_body(dgf_ref, lgf_ref, dgs_ref, lgs_ref, g_ref, be_ref, wf_ref,
                 bf_ref, wat_ref, pred_ref, attn_ref):
    gf = jnp.concatenate([dgf_ref[...], lgf_ref[...]], axis=1)
    mu = jnp.mean(gf, axis=0, keepdims=True)
    var = jnp.mean((gf - mu) ** 2, axis=0, keepdims=True)
    x = (gf - mu) / jnp.sqrt(var + 1e-5) * g_ref[...] + be_ref[...]
    x = jax.nn.relu(jnp.dot(x, wf_ref[...], preferred_element_type=jnp.float32)
                    + bf_ref[...])
    a = jnp.dot(x, wat_ref[...], preferred_element_type=jnp.float32)
    a = jnp.where(a > 0, a, 0.2 * a)
    amax = jnp.max(a, axis=1, keepdims=True)
    ea = jnp.exp(a - amax)
    attn = ea / jnp.sum(ea, axis=1, keepdims=True)
    score = jnp.concatenate([dgs_ref[...], lgs_ref[...]], axis=1)
    pred_ref[...] = jnp.sum(attn * score, axis=1, keepdims=True)
    attn_ref[...] = attn


def _fusion(dgf, lgf, dgs, lgs, gamma, beta, Wf, bf, Wa):
    b, hd = dgf.shape
    h2 = 2 * hd
    pred, attn = pl.pallas_call(
        _fusion_body,
        out_shape=[jax.ShapeDtypeStruct((b, 1), jnp.float32),
                   jax.ShapeDtypeStruct((b, 2), jnp.float32)],
    )(dgf, lgf, dgs, lgs, gamma.reshape(1, h2), beta.reshape(1, h2), Wf,
      bf.reshape(1, h2), Wa.T)
    return pred.reshape(b), attn


def kernel(dg_node_feat_discrete, lg_node_feat_continuous,
           lg_node_feat_discrete, dg_edge_feat, lg_edge_feat, dg_edge_index,
           lg_edge_index, lg_edge_atom, dg_node_seg, lg_node_seg, params):
    p = params
    n_dg = dg_node_feat_discrete.shape[0]
    n_lg = lg_node_feat_discrete.shape[0]
    n_graphs = 64

    dg_disc = dg_node_feat_discrete.astype(jnp.int32)
    lg_disc = lg_node_feat_discrete.astype(jnp.int32)
    dg_src = dg_edge_index[0]
    dg_dst = dg_edge_index[1]
    lg_src = lg_edge_index[0]
    lg_dst = lg_edge_index[1]

    dg_h = _dg_node_module(dg_disc, p['emb_dg'], p['W_dg_node'], p['b_dg_node'])
    dg_eh = _edge_module(dg_edge_feat, p['W_dg_edge'], p['b_dg_edge'], kind=1)
    lg_h = _lg_node_module(lg_disc, lg_node_feat_continuous, p['emb_lg'],
                           p['W_lg_node'], p['b_lg_node'])
    lg_eh = _edge_module(lg_edge_feat, p['W_lg_edge'], p['b_lg_edge'], kind=2)

    nc = 2
    for i in range(nc):
        hsrc = jnp.take(dg_h, dg_src, axis=0)
        m = _linrelu([hsrc, dg_eh, lg_h], p['Wm_dg_%d' % i], p['bm_dg_%d' % i])
        agg = jax.ops.segment_sum(m, dg_dst, num_segments=n_dg)
        dg_h_new = _linrelu([dg_h, agg], p['Wu_dg_%d' % i], p['bu_dg_%d' % i])
        lg_cross = jnp.take(dg_h, lg_edge_atom, axis=0)
        lsrc = jnp.take(lg_h, lg_src, axis=0)
        m2 = _linrelu([lsrc, lg_eh, lg_cross], p['Wm_lg_%d' % i],
                      p['bm_lg_%d' % i])
        agg2 = jax.ops.segment_sum(m2, lg_dst, num_segments=n_lg)
        lg_h = _linrelu([lg_h, agg2], p['Wu_lg_%d' % i], p['bu_lg_%d' % i])
        dg_h = dg_h_new

    dg_feat, dg_score = _readout(dg_h, dg_disc, dg_node_seg.astype(jnp.int32),
                                 p['Wo_dg'], p['bo_dg'], p['sc_dg'],
                                 p['bi_dg'], p['mean_dg'], p['std_dg'],
                                 n_graphs)
    lg_feat, lg_score = _readout(lg_h, lg_disc, lg_node_seg.astype(jnp.int32),
                                 p['Wo_lg'], p['bo_lg'], p['sc_lg'],
                                 p['bi_lg'], p['mean_lg'], p['std_lg'],
                                 n_graphs)
    pred, attn = _fusion(dg_feat, lg_feat, dg_score, lg_score, p['bn_gamma'],
                         p['bn_beta'], p['Wf'], p['bf'], p['Wa'])
    return dg_score, lg_score, pred, attn


# speedup vs baseline: 2.1235x; 1.2665x over previous
import functools
import numpy as np
import jax
import jax.numpy as jnp
from jax import lax
from jax.experimental import pallas as pl
from jax.experimental.pallas import tpu as pltpu, tpu_sc as plsc

_CUT_R = 5.0


def _sc_gather(table, idx):
    b = idx.shape[0]
    d = table.shape[1]
    c = 128
    nchunks = b // c
    info = plsc.get_sparse_core_info()
    nc, ns = info.num_cores, info.num_subcores
    nw = nc * ns
    iters = (nchunks + nw - 1) // nw
    mesh = plsc.VectorSubcoreMesh(core_axis_name="c", subcore_axis_name="s")

    @functools.partial(
        pl.kernel, mesh=mesh,
        compiler_params=pltpu.CompilerParams(use_tc_tiling_on_sc=False),
        out_type=jax.ShapeDtypeStruct((b, d), jnp.float32),
        scratch_types=[pltpu.VMEM((c,), jnp.int32),
                       pltpu.VMEM((c, d), jnp.float32),
                       pltpu.SemaphoreType.DMA],
    )
    def k(table_hbm, idx_hbm, out_hbm, idx_v, rows_v, sem):
        wid = lax.axis_index("s") * nc + lax.axis_index("c")

        def body(j, carry):
            cid = wid + j * nw

            @pl.when(cid < nchunks)
            def _():
                base = cid * c
                pltpu.sync_copy(idx_hbm.at[pl.ds(base, c)], idx_v)
                pltpu.async_copy(table_hbm.at[idx_v], rows_v, sem).wait()
                pltpu.sync_copy(rows_v, out_hbm.at[pl.ds(base, c)])

            return carry

        lax.fori_loop(0, iters, body, 0)

    return k(table, idx.astype(jnp.int32))


def _bn(n, cap=4096):
    best = None
    for b in range(8, cap + 1, 8):
        if n % b == 0:
            best = b
    return best if best is not None else n


def _linrelu_body(nx, *refs):
    xs = refs[:nx]
    w_ref, b_ref, o_ref = refs[nx], refs[nx + 1], refs[nx + 2]
    acc = xs[0][...]
    for r in xs[1:]:
        acc = acc + r[...]
    y = jnp.dot(acc, w_ref[...], preferred_element_type=jnp.float32)
    o_ref[...] = jax.nn.relu(y + b_ref[...])


def _linrelu(xs, W, b):
    n, k = xs[0].shape
    h = W.shape[1]
    bn = _bn(n)
    grid = (n // bn,)
    in_specs = [pl.BlockSpec((bn, k), lambda i: (i, 0)) for _ in xs]
    in_specs += [pl.BlockSpec((k, h), lambda i: (0, 0)),
                 pl.BlockSpec((1, h), lambda i: (0, 0))]
    return pl.pallas_call(
        functools.partial(_linrelu_body, len(xs)),
        grid=grid,
        in_specs=in_specs,
        out_specs=pl.BlockSpec((bn, h), lambda i: (i, 0)),
        out_shape=jax.ShapeDtypeStruct((n, h), jnp.float32),
    )(*xs, W, b.reshape(1, h))


def _rbf_body(kind, r_dim, d_ref, w_ref, b_ref, o_ref):
    d = d_ref[...]
    bn = d.shape[0]
    it = lax.broadcasted_iota(jnp.int32, (bn, r_dim), 1).astype(jnp.float32)
    if kind == 2:
        mu = it * (np.pi / (r_dim - 1))
        gamma = 8.0
    else:
        mu = it * (_CUT_R / (r_dim - 1))
        gamma = 10.0
    feat = jnp.exp(-gamma * (d - mu) ** 2)
    if kind == 1:
        env = 0.5 * (jnp.cos(np.pi * jnp.clip(d, 0.0, _CUT_R) / _CUT_R) + 1.0)
        feat = feat * env
    y = jnp.dot(feat, w_ref[...], preferred_element_type=jnp.float32)
    o_ref[...] = jax.nn.relu(y + b_ref[...])


def _edge_module(d, W, b, kind):
    n = d.shape[0]
    r_dim, h = W.shape
    bn = _bn(n)
    return pl.pallas_call(
        functools.partial(_rbf_body, kind, r_dim),
        grid=(n // bn,),
        in_specs=[pl.BlockSpec((bn, 1), lambda i: (i, 0)),
                  pl.BlockSpec((r_dim, h), lambda i: (0, 0)),
                  pl.BlockSpec((1, h), lambda i: (0, 0))],
        out_specs=pl.BlockSpec((bn, h), lambda i: (i, 0)),
        out_shape=jax.ShapeDtypeStruct((n, h), jnp.float32),
    )(d, W, b.reshape(1, h))


def _dgnode_body(t_dim, disc_ref, emb_ref, w_ref, b_ref, o_ref):
    disc = disc_ref[...]
    bn = disc.shape[0]
    it = lax.broadcasted_iota(jnp.int32, (bn, t_dim), 1)
    oh = (disc == it).astype(jnp.float32)
    e = jnp.dot(oh, emb_ref[...], preferred_element_type=jnp.float32)
    y = jnp.dot(e, w_ref[...], preferred_element_type=jnp.float32)
    o_ref[...] = jax.nn.relu(y + b_ref[...])


def _dg_node_module(disc, emb, W, b):
    n = disc.shape[0]
    t_dim, h = emb.shape
    bn = _bn(n)
    return pl.pallas_call(
        functools.partial(_dgnode_body, t_dim),
        grid=(n // bn,),
        in_specs=[pl.BlockSpec((bn, 1), lambda i: (i, 0)),
                  pl.BlockSpec((t_dim, h), lambda i: (0, 0)),
                  pl.BlockSpec((h, h), lambda i: (0, 0)),
                  pl.BlockSpec((1, h), lambda i: (0, 0))],
        out_specs=pl.BlockSpec((bn, h), lambda i: (i, 0)),
        out_shape=jax.ShapeDtypeStruct((n, h), jnp.float32),
    )(disc.reshape(n, 1), emb, W, b.reshape(1, h))


def _lgnode_body(t_dim, r_dim, disc_ref, cont_ref, emb_ref, w1_ref, w2_ref,
                 b_ref, o_ref):
    disc = disc_ref[...]
    bn = disc.shape[0]
    it = lax.broadcasted_iota(jnp.int32, (bn, t_dim), 1)
    oh = (disc == it).astype(jnp.float32)
    e = jnp.dot(oh, emb_ref[...], preferred_element_type=jnp.float32)
    d = cont_ref[...]
    itf = lax.broadcasted_iota(jnp.int32, (bn, r_dim), 1).astype(jnp.float32)
    mu = itf * (_CUT_R / (r_dim - 1))
    feat = jnp.exp(-10.0 * (d - mu) ** 2)
    y = (jnp.dot(e, w1_ref[...], preferred_element_type=jnp.float32)
         + jnp.dot(feat, w2_ref[...], preferred_element_type=jnp.float32))
    o_ref[...] = jax.nn.relu(y + b_ref[...])


def _lg_node_module(disc, cont, emb, W, b):
    n = disc.shape[0]
    t_dim, h = emb.shape
    r_dim = W.shape[0] - h
    bn = _bn(n)
    return pl.pallas_call(
        functools.partial(_lgnode_body, t_dim, r_dim),
        grid=(n // bn,),
        in_specs=[pl.BlockSpec((bn, 1), lambda i: (i, 0)),
                  pl.BlockSpec((bn, 1), lambda i: (i, 0)),
                  pl.BlockSpec((t_dim, h), lambda i: (0, 0)),
                  pl.BlockSpec((h, h), lambda i: (0, 0)),
                  pl.BlockSpec((r_dim, h), lambda i: (0, 0)),
                  pl.BlockSpec((1, h), lambda i: (0, 0))],
        out_specs=pl.BlockSpec((bn, h), lambda i: (i, 0)),
        out_shape=jax.ShapeDtypeStruct((n, h), jnp.float32),
    )(disc.reshape(n, 1), cont, emb, W[:h], W[h:], b.reshape(1, h))


def _readout_body(t_dim, n_graphs, h_ref, disc_ref, seg_ref, wo_ref, sct_ref,
                  bit_ref, cons_ref, score_ref, feat_ref):
    i = pl.program_id(0)
    h = h_ref[...]
    bn = h.shape[0]
    disc = disc_ref[...]
    it = lax.broadcasted_iota(jnp.int32, (bn, t_dim), 1)
    oh = (disc == it).astype(jnp.float32)
    node_out = jnp.sum(h * wo_ref[...], axis=1, keepdims=True)
    scv = jnp.sum(oh * sct_ref[...], axis=1, keepdims=True)
    biv = jnp.sum(oh * bit_ref[...], axis=1, keepdims=True)
    bo = cons_ref[0, 0]
    std = cons_ref[0, 1]
    mean = cons_ref[0, 2]
    node_out = scv * (node_out + bo) + biv
    node_out = node_out * std + mean
    seg = seg_ref[...]
    itg = lax.broadcasted_iota(jnp.int32, (bn, n_graphs), 1)
    ohs = (seg == itg).astype(jnp.float32)
    part_score = lax.dot_general(ohs, node_out, (((0,), (0,)), ((), ())),
                                 preferred_element_type=jnp.float32)
    part_feat = lax.dot_general(ohs, h, (((0,), (0,)), ((), ())),
                                preferred_element_type=jnp.float32)

    @pl.when(i == 0)
    def _():
        score_ref[...] = jnp.zeros_like(score_ref)
        feat_ref[...] = jnp.zeros_like(feat_ref)

    score_ref[...] += part_score
    feat_ref[...] += part_feat


def _readout(h, disc, seg, Wo, bo, sc, bi, mean, std, n_graphs):
    n, hd = h.shape
    t_dim = sc.shape[0]
    bn = _bn(n)
    cons = jnp.stack([bo[0], std[0], mean[0]]).reshape(1, 3)
    score, feat = pl.pallas_call(
        functools.partial(_readout_body, t_dim, n_graphs),
        grid=(n // bn,),
        in_specs=[pl.BlockSpec((bn, hd), lambda i: (i, 0)),
                  pl.BlockSpec((bn, 1), lambda i: (i, 0)),
                  pl.BlockSpec((bn, 1), lambda i: (i, 0)),
                  pl.BlockSpec((1, hd), lambda i: (0, 0)),
                  pl.BlockSpec((1, t_dim), lambda i: (0, 0)),
                  pl.BlockSpec((1, t_dim), lambda i: (0, 0)),
                  pl.BlockSpec((1, 3), lambda i: (0, 0))],
        out_specs=[pl.BlockSpec((n_graphs, 1), lambda i: (0, 0)),
                   pl.BlockSpec((n_graphs, hd), lambda i: (0, 0))],
        out_shape=[jax.ShapeDtypeStruct((n_graphs, 1), jnp.float32),
                   jax.ShapeDtypeStruct((n_graphs, hd), jnp.float32)],
    )(h, disc.reshape(n, 1), seg.reshape(n, 1), Wo.reshape(1, hd),
      sc.reshape(1, t_dim), bi.reshape(1, t_dim), cons)
    return feat, score


def _fusion_body(dgf_ref, lgf_ref, dgs_ref, lgs_ref, g_ref, be_ref, wf_ref,
                 bf_ref, wat_ref, pred_ref, attn_ref):
    gf = jnp.concatenate([dgf_ref[...], lgf_ref[...]], axis=1)
    mu = jnp.mean(gf, axis=0, keepdims=True)
    var = jnp.mean((gf - mu) ** 2, axis=0, keepdims=True)
    x = (gf - mu) / jnp.sqrt(var + 1e-5) * g_ref[...] + be_ref[...]
    x = jax.nn.relu(jnp.dot(x, wf_ref[...], preferred_element_type=jnp.float32)
                    + bf_ref[...])
    a = jnp.dot(x, wat_ref[...], preferred_element_type=jnp.float32)
    a = jnp.where(a > 0, a, 0.2 * a)
    amax = jnp.max(a, axis=1, keepdims=True)
    ea = jnp.exp(a - amax)
    attn = ea / jnp.sum(ea, axis=1, keepdims=True)
    score = jnp.concatenate([dgs_ref[...], lgs_ref[...]], axis=1)
    pred_ref[...] = jnp.sum(attn * score, axis=1, keepdims=True)
    attn_ref[...] = attn


def _fusion(dgf, lgf, dgs, lgs, gamma, beta, Wf, bf, Wa):
    b, hd = dgf.shape
    h2 = 2 * hd
    pred, attn = pl.pallas_call(
        _fusion_body,
        out_shape=[jax.ShapeDtypeStruct((b, 1), jnp.float32),
                   jax.ShapeDtypeStruct((b, 2), jnp.float32)],
    )(dgf, lgf, dgs, lgs, gamma.reshape(1, h2), beta.reshape(1, h2), Wf,
      bf.reshape(1, h2), Wa.T)
    return pred.reshape(b), attn


def kernel(dg_node_feat_discrete, lg_node_feat_continuous,
           lg_node_feat_discrete, dg_edge_feat, lg_edge_feat, dg_edge_index,
           lg_edge_index, lg_edge_atom, dg_node_seg, lg_node_seg, params):
    p = params
    n_dg = dg_node_feat_discrete.shape[0]
    n_lg = lg_node_feat_discrete.shape[0]
    n_graphs = 64

    dg_disc = dg_node_feat_discrete.astype(jnp.int32)
    lg_disc = lg_node_feat_discrete.astype(jnp.int32)
    dg_src = dg_edge_index[0]
    dg_dst = dg_edge_index[1]
    lg_src = lg_edge_index[0]
    lg_dst = lg_edge_index[1]

    dg_h = _dg_node_module(dg_disc, p['emb_dg'], p['W_dg_node'], p['b_dg_node'])
    dg_eh = _edge_module(dg_edge_feat, p['W_dg_edge'], p['b_dg_edge'], kind=1)
    lg_h = _lg_node_module(lg_disc, lg_node_feat_continuous, p['emb_lg'],
                           p['W_lg_node'], p['b_lg_node'])
    lg_eh = _edge_module(lg_edge_feat, p['W_lg_edge'], p['b_lg_edge'], kind=2)

    nc = 2
    for i in range(nc):
        hsrc = _sc_gather(dg_h, dg_src)
        m = _linrelu([hsrc, dg_eh, lg_h], p['Wm_dg_%d' % i], p['bm_dg_%d' % i])
        agg = jax.ops.segment_sum(m, dg_dst, num_segments=n_dg)
        dg_h_new = _linrelu([dg_h, agg], p['Wu_dg_%d' % i], p['bu_dg_%d' % i])
        lg_cross = _sc_gather(dg_h, lg_edge_atom)
        lsrc = _sc_gather(lg_h, lg_src)
        m2 = _linrelu([lsrc, lg_eh, lg_cross], p['Wm_lg_%d' % i],
                      p['bm_lg_%d' % i])
        agg2 = jax.ops.segment_sum(m2, lg_dst, num_segments=n_lg)
        lg_h = _linrelu([lg_h, agg2], p['Wu_lg_%d' % i], p['bu_lg_%d' % i])
        dg_h = dg_h_new

    dg_feat, dg_score = _readout(dg_h, dg_disc, dg_node_seg.astype(jnp.int32),
                                 p['Wo_dg'], p['bo_dg'], p['sc_dg'],
                                 p['bi_dg'], p['mean_dg'], p['std_dg'],
                                 n_graphs)
    lg_feat, lg_score = _readout(lg_h, lg_disc, lg_node_seg.astype(jnp.int32),
                                 p['Wo_lg'], p['bo_lg'], p['sc_lg'],
                                 p['bi_lg'], p['mean_lg'], p['std_lg'],
                                 n_graphs)
    pred, attn = _fusion(dg_feat, lg_feat, dg_score, lg_score, p['bn_gamma'],
                         p['bn_beta'], p['Wf'], p['bf'], p['Wa'])
    return dg_score, lg_score, pred, attn


# speedup vs baseline: 2.1256x; 1.0010x over previous
import functools
import numpy as np
import jax
import jax.numpy as jnp
from jax import lax
from jax.experimental import pallas as pl
from jax.experimental.pallas import tpu as pltpu, tpu_sc as plsc

_CUT_R = 5.0


def _sc_gather(table, idx):
    b = idx.shape[0]
    d = table.shape[1]
    c = 128
    nchunks = b // c
    info = plsc.get_sparse_core_info()
    nc, ns = info.num_cores, info.num_subcores
    nw = nc * ns
    iters = (nchunks + nw - 1) // nw
    mesh = plsc.VectorSubcoreMesh(core_axis_name="c", subcore_axis_name="s")

    @functools.partial(
        pl.kernel, mesh=mesh,
        compiler_params=pltpu.CompilerParams(use_tc_tiling_on_sc=False),
        out_type=jax.ShapeDtypeStruct((b, d), jnp.float32),
        scratch_types=[pltpu.VMEM((c,), jnp.int32),
                       pltpu.VMEM((c, d), jnp.float32),
                       pltpu.VMEM((c, d), jnp.float32),
                       pltpu.SemaphoreType.DMA,
                       pltpu.SemaphoreType.DMA,
                       pltpu.SemaphoreType.DMA],
    )
    def k(table_hbm, idx_hbm, out_hbm, idx_v, rows0, rows1, gsem, ws0, ws1):
        wid = lax.axis_index("s") * nc + lax.axis_index("c")
        rows = (rows0, rows1)
        wsem = (ws0, ws1)

        def body(jj, carry):
            for b2 in range(2):
                j = jj * 2 + b2
                cid = wid + j * nw

                @pl.when(cid < nchunks)
                def _():
                    base = cid * c
                    pltpu.sync_copy(idx_hbm.at[pl.ds(base, c)], idx_v)
                    pltpu.async_copy(table_hbm.at[idx_v], rows[b2], gsem
                                     ).wait()

                    @pl.when(jj > 0)
                    def _():
                        pltpu.make_async_copy(
                            rows[b2], out_hbm.at[pl.ds(0, c)], wsem[b2]
                        ).wait()

                    pltpu.async_copy(rows[b2], out_hbm.at[pl.ds(base, c)],
                                     wsem[b2])
            return carry

        lax.fori_loop(0, iters // 2, body, 0)
        cnt = (nchunks - wid + nw - 1) // nw

        @pl.when(cnt > 0)
        def _():
            pltpu.make_async_copy(rows0, out_hbm.at[pl.ds(0, c)], ws0).wait()

        @pl.when(cnt > 1)
        def _():
            pltpu.make_async_copy(rows1, out_hbm.at[pl.ds(0, c)], ws1).wait()

    return k(table, idx.astype(jnp.int32))


def _bn(n, cap=4096):
    best = None
    for b in range(8, cap + 1, 8):
        if n % b == 0:
            best = b
    return best if best is not None else n


def _linrelu_body(nx, *refs):
    xs = refs[:nx]
    w_ref, b_ref, o_ref = refs[nx], refs[nx + 1], refs[nx + 2]
    acc = xs[0][...]
    for r in xs[1:]:
        acc = acc + r[...]
    y = jnp.dot(acc, w_ref[...], preferred_element_type=jnp.float32)
    o_ref[...] = jax.nn.relu(y + b_ref[...])


def _linrelu(xs, W, b):
    n, k = xs[0].shape
    h = W.shape[1]
    bn = _bn(n)
    grid = (n // bn,)
    in_specs = [pl.BlockSpec((bn, k), lambda i: (i, 0)) for _ in xs]
    in_specs += [pl.BlockSpec((k, h), lambda i: (0, 0)),
                 pl.BlockSpec((1, h), lambda i: (0, 0))]
    return pl.pallas_call(
        functools.partial(_linrelu_body, len(xs)),
        grid=grid,
        in_specs=in_specs,
        out_specs=pl.BlockSpec((bn, h), lambda i: (i, 0)),
        out_shape=jax.ShapeDtypeStruct((n, h), jnp.float32),
    )(*xs, W, b.reshape(1, h))


def _rbf_body(kind, r_dim, d_ref, w_ref, b_ref, o_ref):
    d = d_ref[...]
    bn = d.shape[0]
    it = lax.broadcasted_iota(jnp.int32, (bn, r_dim), 1).astype(jnp.float32)
    if kind == 2:
        mu = it * (np.pi / (r_dim - 1))
        gamma = 8.0
    else:
        mu = it * (_CUT_R / (r_dim - 1))
        gamma = 10.0
    feat = jnp.exp(-gamma * (d - mu) ** 2)
    if kind == 1:
        env = 0.5 * (jnp.cos(np.pi * jnp.clip(d, 0.0, _CUT_R) / _CUT_R) + 1.0)
        feat = feat * env
    y = jnp.dot(feat, w_ref[...], preferred_element_type=jnp.float32)
    o_ref[...] = jax.nn.relu(y + b_ref[...])


def _edge_module(d, W, b, kind):
    n = d.shape[0]
    r_dim, h = W.shape
    bn = _bn(n)
    return pl.pallas_call(
        functools.partial(_rbf_body, kind, r_dim),
        grid=(n // bn,),
        in_specs=[pl.BlockSpec((bn, 1), lambda i: (i, 0)),
                  pl.BlockSpec((r_dim, h), lambda i: (0, 0)),
                  pl.BlockSpec((1, h), lambda i: (0, 0))],
        out_specs=pl.BlockSpec((bn, h), lambda i: (i, 0)),
        out_shape=jax.ShapeDtypeStruct((n, h), jnp.float32),
    )(d, W, b.reshape(1, h))


def _dgnode_body(t_dim, disc_ref, emb_ref, w_ref, b_ref, o_ref):
    disc = disc_ref[...]
    bn = disc.shape[0]
    it = lax.broadcasted_iota(jnp.int32, (bn, t_dim), 1)
    oh = (disc == it).astype(jnp.float32)
    e = jnp.dot(oh, emb_ref[...], preferred_element_type=jnp.float32)
    y = jnp.dot(e, w_ref[...], preferred_element_type=jnp.float32)
    o_ref[...] = jax.nn.relu(y + b_ref[...])


def _dg_node_module(disc, emb, W, b):
    n = disc.shape[0]
    t_dim, h = emb.shape
    bn = _bn(n)
    return pl.pallas_call(
        functools.partial(_dgnode_body, t_dim),
        grid=(n // bn,),
        in_specs=[pl.BlockSpec((bn, 1), lambda i: (i, 0)),
                  pl.BlockSpec((t_dim, h), lambda i: (0, 0)),
                  pl.BlockSpec((h, h), lambda i: (0, 0)),
                  pl.BlockSpec((1, h), lambda i: (0, 0))],
        out_specs=pl.BlockSpec((bn, h), lambda i: (i, 0)),
        out_shape=jax.ShapeDtypeStruct((n, h), jnp.float32),
    )(disc.reshape(n, 1), emb, W, b.reshape(1, h))


def _lgnode_body(t_dim, r_dim, disc_ref, cont_ref, emb_ref, w1_ref, w2_ref,
                 b_ref, o_ref):
    disc = disc_ref[...]
    bn = disc.shape[0]
    it = lax.broadcasted_iota(jnp.int32, (bn, t_dim), 1)
    oh = (disc == it).astype(jnp.float32)
    e = jnp.dot(oh, emb_ref[...], preferred_element_type=jnp.float32)
    d = cont_ref[...]
    itf = lax.broadcasted_iota(jnp.int32, (bn, r_dim), 1).astype(jnp.float32)
    mu = itf * (_CUT_R / (r_dim - 1))
    feat = jnp.exp(-10.0 * (d - mu) ** 2)
    y = (jnp.dot(e, w1_ref[...], preferred_element_type=jnp.float32)
         + jnp.dot(feat, w2_ref[...], preferred_element_type=jnp.float32))
    o_ref[...] = jax.nn.relu(y + b_ref[...])


def _lg_node_module(disc, cont, emb, W, b):
    n = disc.shape[0]
    t_dim, h = emb.shape
    r_dim = W.shape[0] - h
    bn = _bn(n)
    return pl.pallas_call(
        functools.partial(_lgnode_body, t_dim, r_dim),
        grid=(n // bn,),
        in_specs=[pl.BlockSpec((bn, 1), lambda i: (i, 0)),
                  pl.BlockSpec((bn, 1), lambda i: (i, 0)),
                  pl.BlockSpec((t_dim, h), lambda i: (0, 0)),
                  pl.BlockSpec((h, h), lambda i: (0, 0)),
                  pl.BlockSpec((r_dim, h), lambda i: (0, 0)),
                  pl.BlockSpec((1, h), lambda i: (0, 0))],
        out_specs=pl.BlockSpec((bn, h), lambda i: (i, 0)),
        out_shape=jax.ShapeDtypeStruct((n, h), jnp.float32),
    )(disc.reshape(n, 1), cont, emb, W[:h], W[h:], b.reshape(1, h))


def _readout_body(t_dim, n_graphs, h_ref, disc_ref, seg_ref, wo_ref, sct_ref,
                  bit_ref, cons_ref, score_ref, feat_ref):
    i = pl.program_id(0)
    h = h_ref[...]
    bn = h.shape[0]
    disc = disc_ref[...]
    it = lax.broadcasted_iota(jnp.int32, (bn, t_dim), 1)
    oh = (disc == it).astype(jnp.float32)
    node_out = jnp.sum(h * wo_ref[...], axis=1, keepdims=True)
    scv = jnp.sum(oh * sct_ref[...], axis=1, keepdims=True)
    biv = jnp.sum(oh * bit_ref[...], axis=1, keepdims=True)
    bo = cons_ref[0, 0]
    std = cons_ref[0, 1]
    mean = cons_ref[0, 2]
    node_out = scv * (node_out + bo) + biv
    node_out = node_out * std + mean
    seg = seg_ref[...]
    itg = lax.broadcasted_iota(jnp.int32, (bn, n_graphs), 1)
    ohs = (seg == itg).astype(jnp.float32)
    part_score = lax.dot_general(ohs, node_out, (((0,), (0,)), ((), ())),
                                 preferred_element_type=jnp.float32)
    part_feat = lax.dot_general(ohs, h, (((0,), (0,)), ((), ())),
                                preferred_element_type=jnp.float32)

    @pl.when(i == 0)
    def _():
        score_ref[...] = jnp.zeros_like(score_ref)
        feat_ref[...] = jnp.zeros_like(feat_ref)

    score_ref[...] += part_score
    feat_ref[...] += part_feat


def _readout(h, disc, seg, Wo, bo, sc, bi, mean, std, n_graphs):
    n, hd = h.shape
    t_dim = sc.shape[0]
    bn = _bn(n)
    cons = jnp.stack([bo[0], std[0], mean[0]]).reshape(1, 3)
    score, feat = pl.pallas_call(
        functools.partial(_readout_body, t_dim, n_graphs),
        grid=(n // bn,),
        in_specs=[pl.BlockSpec((bn, hd), lambda i: (i, 0)),
                  pl.BlockSpec((bn, 1), lambda i: (i, 0)),
                  pl.BlockSpec((bn, 1), lambda i: (i, 0)),
                  pl.BlockSpec((1, hd), lambda i: (0, 0)),
                  pl.BlockSpec((1, t_dim), lambda i: (0, 0)),
                  pl.BlockSpec((1, t_dim), lambda i: (0, 0)),
                  pl.BlockSpec((1, 3), lambda i: (0, 0))],
        out_specs=[pl.BlockSpec((n_graphs, 1), lambda i: (0, 0)),
                   pl.BlockSpec((n_graphs, hd), lambda i: (0, 0))],
        out_shape=[jax.ShapeDtypeStruct((n_graphs, 1), jnp.float32),
                   jax.ShapeDtypeStruct((n_graphs, hd), jnp.float32)],
    )(h, disc.reshape(n, 1), seg.reshape(n, 1), Wo.reshape(1, hd),
      sc.reshape(1, t_dim), bi.reshape(1, t_dim), cons)
    return feat, score


def _fusion_body(dgf_ref, lgf_ref, dgs_ref, lgs_ref, g_ref, be_ref, wf_ref,
                 bf_ref, wat_ref, pred_ref, attn_ref):
    gf = jnp.concatenate([dgf_ref[...], lgf_ref[...]], axis=1)
    mu = jnp.mean(gf, axis=0, keepdims=True)
    var = jnp.mean((gf - mu) ** 2, axis=0, keepdims=True)
    x = (gf - mu) / jnp.sqrt(var + 1e-5) * g_ref[...] + be_ref[...]
    x = jax.nn.relu(jnp.dot(x, wf_ref[...], preferred_element_type=jnp.float32)
                    + bf_ref[...])
    a = jnp.dot(x, wat_ref[...], preferred_element_type=jnp.float32)
    a = jnp.where(a > 0, a, 0.2 * a)
    amax = jnp.max(a, axis=1, keepdims=True)
    ea = jnp.exp(a - amax)
    attn = ea / jnp.sum(ea, axis=1, keepdims=True)
    score = jnp.concatenate([dgs_ref[...], lgs_ref[...]], axis=1)
    pred_ref[...] = jnp.sum(attn * score, axis=1, keepdims=True)
    attn_ref[...] = attn


def _fusion(dgf, lgf, dgs, lgs, gamma, beta, Wf, bf, Wa):
    b, hd = dgf.shape
    h2 = 2 * hd
    pred, attn = pl.pallas_call(
        _fusion_body,
        out_shape=[jax.ShapeDtypeStruct((b, 1), jnp.float32),
                   jax.ShapeDtypeStruct((b, 2), jnp.float32)],
    )(dgf, lgf, dgs, lgs, gamma.reshape(1, h2), beta.reshape(1, h2), Wf,
      bf.reshape(1, h2), Wa.T)
    return pred.reshape(b), attn


def kernel(dg_node_feat_discrete, lg_node_feat_continuous,
           lg_node_feat_discrete, dg_edge_feat, lg_edge_feat, dg_edge_index,
           lg_edge_index, lg_edge_atom, dg_node_seg, lg_node_seg, params):
    p = params
    n_dg = dg_node_feat_discrete.shape[0]
    n_lg = lg_node_feat_discrete.shape[0]
    n_graphs = 64

    dg_disc = dg_node_feat_discrete.astype(jnp.int32)
    lg_disc = lg_node_feat_discrete.astype(jnp.int32)
    dg_src = dg_edge_index[0]
    dg_dst = dg_edge_index[1]
    lg_src = lg_edge_index[0]
    lg_dst = lg_edge_index[1]

    dg_h = _dg_node_module(dg_disc, p['emb_dg'], p['W_dg_node'], p['b_dg_node'])
    dg_eh = _edge_module(dg_edge_feat, p['W_dg_edge'], p['b_dg_edge'], kind=1)
    lg_h = _lg_node_module(lg_disc, lg_node_feat_continuous, p['emb_lg'],
                           p['W_lg_node'], p['b_lg_node'])
    lg_eh = _edge_module(lg_edge_feat, p['W_lg_edge'], p['b_lg_edge'], kind=2)

    nc = 2
    for i in range(nc):
        hsrc = _sc_gather(dg_h, dg_src)
        m = _linrelu([hsrc, dg_eh, lg_h], p['Wm_dg_%d' % i], p['bm_dg_%d' % i])
        agg = jax.ops.segment_sum(m, dg_dst, num_segments=n_dg)
        dg_h_new = _linrelu([dg_h, agg], p['Wu_dg_%d' % i], p['bu_dg_%d' % i])
        lg_cross = _sc_gather(dg_h, lg_edge_atom)
        lsrc = _sc_gather(lg_h, lg_src)
        m2 = _linrelu([lsrc, lg_eh, lg_cross], p['Wm_lg_%d' % i],
                      p['bm_lg_%d' % i])
        agg2 = jax.ops.segment_sum(m2, lg_dst, num_segments=n_lg)
        lg_h = _linrelu([lg_h, agg2], p['Wu_lg_%d' % i], p['bu_lg_%d' % i])
        dg_h = dg_h_new

    dg_feat, dg_score = _readout(dg_h, dg_disc, dg_node_seg.astype(jnp.int32),
                                 p['Wo_dg'], p['bo_dg'], p['sc_dg'],
                                 p['bi_dg'], p['mean_dg'], p['std_dg'],
                                 n_graphs)
    lg_feat, lg_score = _readout(lg_h, lg_disc, lg_node_seg.astype(jnp.int32),
                                 p['Wo_lg'], p['bo_lg'], p['sc_lg'],
                                 p['bi_lg'], p['mean_lg'], p['std_lg'],
                                 n_graphs)
    pred, attn = _fusion(dg_feat, lg_feat, dg_score, lg_score, p['bn_gamma'],
                         p['bn_beta'], p['Wf'], p['bf'], p['Wa'])
    return dg_score, lg_score, pred, attn
